# Initial kernel scaffold; baseline (speedup 1.0000x reference)
import jax
import jax.numpy as jnp
from jax.experimental import pallas as pl


def kernel(x, edge_index, batch, W1, a_src1, a_dst1, b1, W2, a_src2, a_dst2, b2, Wlin, blin):
    raise NotImplementedError("write your pallas kernel here")


# speedup vs baseline: 1.9551x; 1.9551x over previous
import functools

import jax
import jax.numpy as jnp
from jax import lax
from jax.experimental import pallas as pl
from jax.experimental.pallas import tpu as pltpu
from jax.experimental.pallas import tpu_sc as plsc

_N = 10000
_E = 320000
_G = 32
_D_IN = 128
_H1 = 128
_H2 = 64
_OUT = 10
_NPAD = 10240
_BT = 1024
_NW = 32
_EPT = _E // _NW
_CH = 80
_NCH = _EPT // _CH


def _t1_body(x_ref, w_ref, asr_ref, adr_ref, h_ref, as_ref, ad_ref, cmax_ref):
    i = pl.program_id(0)
    h = jnp.dot(x_ref[...], w_ref[...], preferred_element_type=jnp.float32)
    h_ref[...] = h
    a_s = jnp.sum(h * asr_ref[...], axis=1)
    a_d = jnp.sum(h * adr_ref[...], axis=1)
    as_ref[...] = a_s.reshape(_BT // 128, 128)
    ad_ref[...] = a_d.reshape(_BT // 128, 128)

    @pl.when(i == 0)
    def _():
        cmax_ref[...] = jnp.full((8, 128), -1e30, jnp.float32)

    mas = jnp.max(a_s)
    mad = jnp.max(a_d)
    upd = jnp.concatenate(
        [jnp.full((1, 128), mas), jnp.full((1, 128), mad),
         jnp.full((6, 128), -1e30)], axis=0)
    cmax_ref[...] = jnp.maximum(cmax_ref[...], upd)


def _t1(x_pad, W, a_src, a_dst, H):
    n_blk = _NPAD // _BT
    return pl.pallas_call(
        _t1_body,
        grid=(n_blk,),
        in_specs=[
            pl.BlockSpec((_BT, _D_IN if H == _H1 else _H1), lambda i: (i, 0)),
            pl.BlockSpec((_D_IN if H == _H1 else _H1, H), lambda i: (0, 0)),
            pl.BlockSpec((1, H), lambda i: (0, 0)),
            pl.BlockSpec((1, H), lambda i: (0, 0)),
        ],
        out_specs=[
            pl.BlockSpec((_BT, H), lambda i: (i, 0)),
            pl.BlockSpec((_BT // 128, 128), lambda i: (i, 0)),
            pl.BlockSpec((_BT // 128, 128), lambda i: (i, 0)),
            pl.BlockSpec((8, 128), lambda i: (0, 0)),
        ],
        out_shape=[
            jax.ShapeDtypeStruct((_NPAD, H), jnp.float32),
            jax.ShapeDtypeStruct((_NPAD // 128, 128), jnp.float32),
            jax.ShapeDtypeStruct((_NPAD // 128, 128), jnp.float32),
            jax.ShapeDtypeStruct((8, 128), jnp.float32),
        ],
    )(x_pad, W, a_src.reshape(1, H), a_dst.reshape(1, H))


def _t2_body(acc_ref, den_ref, b_ref, w_ref, asr_ref, adr_ref,
             h_ref, as_ref, ad_ref, cmax_ref):
    i = pl.program_id(0)
    accs = acc_ref[0] + acc_ref[1]
    den = jnp.sum(den_ref[0] + den_ref[1], axis=1, keepdims=True)
    h1 = jnp.maximum(accs / jnp.maximum(den, 1e-16) + b_ref[...], 0.0)
    h2 = jnp.dot(h1, w_ref[...], preferred_element_type=jnp.float32)
    h_ref[...] = h2
    a_s = jnp.sum(h2 * asr_ref[...], axis=1)
    a_d = jnp.sum(h2 * adr_ref[...], axis=1)
    as_ref[...] = a_s.reshape(_BT // 128, 128)
    ad_ref[...] = a_d.reshape(_BT // 128, 128)

    @pl.when(i == 0)
    def _():
        cmax_ref[...] = jnp.full((8, 128), -1e30, jnp.float32)

    mas = jnp.max(a_s)
    mad = jnp.max(a_d)
    upd = jnp.concatenate(
        [jnp.full((1, 128), mas), jnp.full((1, 128), mad),
         jnp.full((6, 128), -1e30)], axis=0)
    cmax_ref[...] = jnp.maximum(cmax_ref[...], upd)


def _t2(acc, den, b1, W2, a_src2, a_dst2):
    n_blk = _NPAD // _BT
    return pl.pallas_call(
        _t2_body,
        grid=(n_blk,),
        in_specs=[
            pl.BlockSpec((2, _BT, _H1), lambda i: (0, i, 0)),
            pl.BlockSpec((2, _BT, 16), lambda i: (0, i, 0)),
            pl.BlockSpec((1, _H1), lambda i: (0, 0)),
            pl.BlockSpec((_H1, _H2), lambda i: (0, 0)),
            pl.BlockSpec((1, _H2), lambda i: (0, 0)),
            pl.BlockSpec((1, _H2), lambda i: (0, 0)),
        ],
        out_specs=[
            pl.BlockSpec((_BT, _H2), lambda i: (i, 0)),
            pl.BlockSpec((_BT // 128, 128), lambda i: (i, 0)),
            pl.BlockSpec((_BT // 128, 128), lambda i: (i, 0)),
            pl.BlockSpec((8, 128), lambda i: (0, 0)),
        ],
        out_shape=[
            jax.ShapeDtypeStruct((_NPAD, _H2), jnp.float32),
            jax.ShapeDtypeStruct((_NPAD // 128, 128), jnp.float32),
            jax.ShapeDtypeStruct((_NPAD // 128, 128), jnp.float32),
            jax.ShapeDtypeStruct((8, 128), jnp.float32),
        ],
    )(acc, den, b1.reshape(1, _H1), W2,
      a_src2.reshape(1, _H2), a_dst2.reshape(1, _H2))


def _t3_body(acc_ref, den_ref, b_ref, batch_ref, wl_ref, bl_ref,
             out_ref, p_s, c_s):
    i = pl.program_id(0)
    accs = acc_ref[0] + acc_ref[1]
    den = jnp.sum(den_ref[0] + den_ref[1], axis=1, keepdims=True)
    h2 = jnp.maximum(accs / jnp.maximum(den, 1e-16) + b_ref[...], 0.0)
    m = batch_ref[...]
    onehot = (m == lax.broadcasted_iota(jnp.int32, (1, _G), 1)
              ).astype(jnp.float32)

    @pl.when(i == 0)
    def _():
        p_s[...] = jnp.zeros((_G, _H2), jnp.float32)
        c_s[...] = jnp.zeros((_G, 128), jnp.float32)

    p_s[...] += lax.dot_general(onehot, h2, (((0,), (0,)), ((), ())),
                                preferred_element_type=jnp.float32)
    c_s[...] += lax.dot_general(onehot, jnp.ones((_BT, 128), jnp.float32),
                                (((0,), (0,)), ((), ())),
                                preferred_element_type=jnp.float32)

    @pl.when(i == pl.num_programs(0) - 1)
    def _():
        cnt = c_s[:, 0:1]
        pooled = p_s[...] / jnp.maximum(cnt, 1.0)
        out_ref[...] = jnp.dot(pooled, wl_ref[...],
                               preferred_element_type=jnp.float32) + bl_ref[...]


def _t3(acc2, den2, b2, batch_pad, Wlin, blin):
    n_blk = _NPAD // _BT
    return pl.pallas_call(
        _t3_body,
        grid=(n_blk,),
        in_specs=[
            pl.BlockSpec((2, _BT, _H2), lambda i: (0, i, 0)),
            pl.BlockSpec((2, _BT, 16), lambda i: (0, i, 0)),
            pl.BlockSpec((1, _H2), lambda i: (0, 0)),
            pl.BlockSpec((_BT, 1), lambda i: (i, 0)),
            pl.BlockSpec((_H2, _OUT), lambda i: (0, 0)),
            pl.BlockSpec((1, _OUT), lambda i: (0, 0)),
        ],
        out_specs=pl.BlockSpec((_G, _OUT), lambda i: (0, 0)),
        out_shape=jax.ShapeDtypeStruct((_G, _OUT), jnp.float32),
        scratch_shapes=[
            pltpu.VMEM((_G, _H2), jnp.float32),
            pltpu.VMEM((_G, 128), jnp.float32),
        ],
    )(acc2, den2, b2.reshape(1, _H2), batch_pad.reshape(_NPAD, 1),
      Wlin, blin.reshape(1, _OUT))


def _edge_phase_jnp(h, src, dst, asv, adv, cmax):
    c = jnp.maximum(cmax[0, 0] + cmax[1, 0],
                    0.2 * (cmax[0, 0] + cmax[1, 0]))
    alpha = asv[src] + adv[dst]
    alpha = jnp.where(alpha >= 0, alpha, 0.2 * alpha)
    ex = jnp.exp(alpha - c)
    den = jax.ops.segment_sum(ex, dst, num_segments=_NPAD)
    acc = jax.ops.segment_sum(h[src] * ex[:, None], dst, num_segments=_NPAD)
    accs = jnp.zeros((2, _NPAD, h.shape[1]), jnp.float32).at[0].set(acc)
    dens = jnp.zeros((2, _NPAD, 16), jnp.float32).at[0, :, 0].set(den)
    return accs, dens


def kernel(x, edge_index, batch, W1, a_src1, a_dst1, b1,
           W2, a_src2, a_dst2, b2, Wlin, blin):
    x_pad = jnp.zeros((_NPAD, _D_IN), jnp.float32).at[:_N].set(x)
    src = edge_index[0].astype(jnp.int32)
    dst = edge_index[1].astype(jnp.int32)
    batch_pad = jnp.concatenate(
        [batch.astype(jnp.int32), jnp.full((_NPAD - _N,), _G, jnp.int32)])

    h1, as1, ad1, cmax1 = _t1(x_pad, W1, a_src1, a_dst1, _H1)
    acc1, den1 = _edge_phase_jnp(h1, src, dst,
                                 as1.reshape(_NPAD), ad1.reshape(_NPAD), cmax1)
    h2, as2, ad2, cmax2 = _t2(acc1, den1, b1, W2, a_src2, a_dst2)
    acc2, den2 = _edge_phase_jnp(h2, src, dst,
                                 as2.reshape(_NPAD), ad2.reshape(_NPAD), cmax2)
    return _t3(acc2, den2, b2, batch_pad, Wlin, blin)


# speedup vs baseline: 3.4373x; 1.7581x over previous
import functools

import jax
import jax.numpy as jnp
from jax import lax
from jax.experimental import pallas as pl
from jax.experimental.pallas import tpu as pltpu
from jax.experimental.pallas import tpu_sc as plsc

_N = 10000
_E = 320000
_G = 32
_D_IN = 128
_H1 = 128
_H2 = 64
_OUT = 10
_NPAD = 10240
_BT = 1024
_NSUB = 8
_CH = 32
_EPT = _E // _NSUB
_NCH = _EPT // _CH
_NR = _NPAD // 2
_NRP = _NR + 128


def _logit_block(h, asr, adr, as_ref, ad_ref, cmax_ref, i):
    a_s = jnp.sum(h * asr, axis=1)
    a_d = jnp.sum(h * adr, axis=1)
    as_ref[...] = a_s.reshape(_BT // 128, 128)
    ad_ref[...] = a_d.reshape(_BT // 128, 128)

    @pl.when(i == 0)
    def _():
        cmax_ref[...] = jnp.full((8, 128), -1e30, jnp.float32)

    upd = jnp.concatenate(
        [jnp.full((1, 128), jnp.max(a_s)), jnp.full((1, 128), jnp.max(a_d)),
         jnp.full((6, 128), -1e30)], axis=0)
    cmax_ref[...] = jnp.maximum(cmax_ref[...], upd)


def _t1_body(x_ref, w_ref, asr_ref, adr_ref, h_ref, as_ref, ad_ref, cmax_ref):
    i = pl.program_id(0)
    h = jnp.dot(x_ref[...], w_ref[...], preferred_element_type=jnp.float32)
    h_ref[...] = h
    _logit_block(h, asr_ref[...], adr_ref[...], as_ref, ad_ref, cmax_ref, i)


def _t1(x_pad, W1, a_src, a_dst):
    n_blk = _NPAD // _BT
    H = _H1
    return pl.pallas_call(
        _t1_body,
        grid=(n_blk,),
        in_specs=[
            pl.BlockSpec((_BT, _D_IN), lambda i: (i, 0)),
            pl.BlockSpec((_D_IN, H), lambda i: (0, 0)),
            pl.BlockSpec((1, H), lambda i: (0, 0)),
            pl.BlockSpec((1, H), lambda i: (0, 0)),
        ],
        out_specs=[
            pl.BlockSpec((_BT, H), lambda i: (i, 0)),
            pl.BlockSpec((_BT // 128, 128), lambda i: (i, 0)),
            pl.BlockSpec((_BT // 128, 128), lambda i: (i, 0)),
            pl.BlockSpec((8, 128), lambda i: (0, 0)),
        ],
        out_shape=[
            jax.ShapeDtypeStruct((_NPAD, H), jnp.float32),
            jax.ShapeDtypeStruct((_NPAD // 128, 128), jnp.float32),
            jax.ShapeDtypeStruct((_NPAD // 128, 128), jnp.float32),
            jax.ShapeDtypeStruct((8, 128), jnp.float32),
        ],
    )(x_pad, W1, a_src.reshape(1, H), a_dst.reshape(1, H))


def _t2_body(acc_ref, den_ref, b_ref, w_ref, asr_ref, adr_ref,
             h_ref, as_ref, ad_ref, cmax_ref):
    i = pl.program_id(0)
    den = jnp.sum(den_ref[...], axis=1, keepdims=True)
    h1 = jnp.maximum(acc_ref[...] / jnp.maximum(den, 1e-16) + b_ref[...], 0.0)
    h2 = jnp.dot(h1, w_ref[...], preferred_element_type=jnp.float32)
    h_ref[...] = jnp.concatenate(
        [h2, jnp.zeros((_BT, _H1 - _H2), jnp.float32)], axis=1)
    _logit_block(h2, asr_ref[...], adr_ref[...], as_ref, ad_ref, cmax_ref, i)


def _t2(acc, den, b1, W2, a_src2, a_dst2):
    n_blk = _NPAD // _BT
    return pl.pallas_call(
        _t2_body,
        grid=(n_blk,),
        in_specs=[
            pl.BlockSpec((_BT, _H1), lambda i: (i, 0)),
            pl.BlockSpec((_BT, 16), lambda i: (i, 0)),
            pl.BlockSpec((1, _H1), lambda i: (0, 0)),
            pl.BlockSpec((_H1, _H2), lambda i: (0, 0)),
            pl.BlockSpec((1, _H2), lambda i: (0, 0)),
            pl.BlockSpec((1, _H2), lambda i: (0, 0)),
        ],
        out_specs=[
            pl.BlockSpec((_BT, _H1), lambda i: (i, 0)),
            pl.BlockSpec((_BT // 128, 128), lambda i: (i, 0)),
            pl.BlockSpec((_BT // 128, 128), lambda i: (i, 0)),
            pl.BlockSpec((8, 128), lambda i: (0, 0)),
        ],
        out_shape=[
            jax.ShapeDtypeStruct((_NPAD, _H1), jnp.float32),
            jax.ShapeDtypeStruct((_NPAD // 128, 128), jnp.float32),
            jax.ShapeDtypeStruct((_NPAD // 128, 128), jnp.float32),
            jax.ShapeDtypeStruct((8, 128), jnp.float32),
        ],
    )(acc, den, b1.reshape(1, _H1), W2,
      a_src2.reshape(1, _H2), a_dst2.reshape(1, _H2))


def _t3_body(acc_ref, den_ref, b_ref, batch_ref, wl_ref, bl_ref,
             out_ref, p_s, c_s):
    i = pl.program_id(0)
    den = jnp.sum(den_ref[...], axis=1, keepdims=True)
    h2 = jnp.maximum(
        acc_ref[:, :_H2] / jnp.maximum(den, 1e-16) + b_ref[...], 0.0)
    m = batch_ref[...]
    onehot = (m == lax.broadcasted_iota(jnp.int32, (1, _G), 1)
              ).astype(jnp.float32)

    @pl.when(i == 0)
    def _():
        p_s[...] = jnp.zeros((_G, _H2), jnp.float32)
        c_s[...] = jnp.zeros((_G, 128), jnp.float32)

    p_s[...] += lax.dot_general(onehot, h2, (((0,), (0,)), ((), ())),
                                preferred_element_type=jnp.float32)
    c_s[...] += lax.dot_general(onehot, jnp.ones((_BT, 128), jnp.float32),
                                (((0,), (0,)), ((), ())),
                                preferred_element_type=jnp.float32)

    @pl.when(i == pl.num_programs(0) - 1)
    def _():
        cnt = c_s[:, 0:1]
        pooled = p_s[...] / jnp.maximum(cnt, 1.0)
        out_ref[...] = jnp.dot(pooled, wl_ref[...],
                               preferred_element_type=jnp.float32) + bl_ref[...]


def _t3(acc2, den2, b2, batch_pad, Wlin, blin):
    n_blk = _NPAD // _BT
    return pl.pallas_call(
        _t3_body,
        grid=(n_blk,),
        in_specs=[
            pl.BlockSpec((_BT, _H1), lambda i: (i, 0)),
            pl.BlockSpec((_BT, 16), lambda i: (i, 0)),
            pl.BlockSpec((1, _H2), lambda i: (0, 0)),
            pl.BlockSpec((_BT, 1), lambda i: (i, 0)),
            pl.BlockSpec((_H2, _OUT), lambda i: (0, 0)),
            pl.BlockSpec((1, _OUT), lambda i: (0, 0)),
        ],
        out_specs=pl.BlockSpec((_G, _OUT), lambda i: (0, 0)),
        out_shape=jax.ShapeDtypeStruct((_G, _OUT), jnp.float32),
        scratch_shapes=[
            pltpu.VMEM((_G, _H2), jnp.float32),
            pltpu.VMEM((_G, 128), jnp.float32),
        ],
    )(acc2, den2, b2.reshape(1, _H2), batch_pad.reshape(_NPAD, 1),
      Wlin, blin.reshape(1, _OUT))


def _make_sc_edge():
    mesh = plsc.VectorSubcoreMesh(core_axis_name="c", subcore_axis_name="s",
                                  num_subcores=_NSUB)
    CR = _NR // _NSUB

    @functools.partial(
        pl.kernel,
        out_type=[
            jax.ShapeDtypeStruct((_NPAD, _H1), jnp.float32),
            jax.ShapeDtypeStruct((_NPAD, 16), jnp.float32),
        ],
        mesh=mesh,
        compiler_params=pltpu.CompilerParams(needs_layout_passes=False),
        scratch_types=[
            pltpu.VMEM_SHARED((_NR, _H1), jnp.float32),
            pltpu.VMEM_SHARED((_NR, 16), jnp.float32),
            pltpu.VMEM((_CH,), jnp.int32),
            pltpu.VMEM((_CH,), jnp.int32),
            pltpu.VMEM((_NPAD // 128, 128), jnp.float32),
            pltpu.VMEM((_NPAD // 128, 128), jnp.float32),
            pltpu.VMEM((16,), jnp.float32),
            pltpu.VMEM((16,), jnp.float32),
            pltpu.VMEM((_CH, _H1), jnp.float32),
            pltpu.VMEM((_CH,), jnp.float32),
            pltpu.VMEM((_CH, 16), jnp.float32),
            pltpu.SMEM((2,), jnp.int32),
            pltpu.SemaphoreType.DMA,
        ],
    )
    def f(h_hbm, srcf, dstf, asad, cflat, acc_out, den_out,
          acc_sh, den_sh, sbuf, dbuf, asv, adv, ca, cd, rows, exc, exb,
          cnt, sem):
        ci = lax.axis_index("c")
        si = lax.axis_index("s")

        @pl.when(si == 0)
        def _():
            cnt[0] = 0
            cnt[1] = 0

        def sc_barrier(phase):
            plsc.fetch_and_add(cnt.at[phase], 1, subcore_id=0)
            lax.while_loop(
                lambda v: v < _NSUB,
                lambda v: plsc.fetch_and_add(cnt.at[phase], 0, subcore_id=0),
                0)

        pltpu.sync_copy(asad.at[0], asv)
        pltpu.sync_copy(asad.at[1], adv)
        pltpu.sync_copy(cflat.at[pl.ds(0, 16)], ca)
        pltpu.sync_copy(cflat.at[pl.ds(128, 16)], cd)
        s = ca[...] + cd[...]
        cvec = jnp.where(s >= 0.0, s, 0.2 * s)
        lane0 = jnp.where(lax.iota(jnp.int32, 16) == 0, 1.0, 0.0)
        z16 = jnp.zeros((16,), jnp.float32)
        off = ci * _NR

        def zrow_body(r, carry):
            for k in range(_H1 // 16):
                rows[r, pl.ds(k * 16, 16)] = z16
            exb[r, pl.ds(0, 16)] = z16
            return carry

        lax.fori_loop(0, _CH, zrow_body, 0)
        base = si * CR
        for j in range(CR // _CH):
            pltpu.sync_copy(rows, acc_sh.at[pl.ds(base + j * _CH, _CH)])
            pltpu.sync_copy(exb, den_sh.at[pl.ds(base + j * _CH, _CH)])
        sc_barrier(0)

        def chunk(cidx, carry):
            ebase = (si * _NCH + cidx) * _CH
            pltpu.sync_copy(srcf.at[pl.ds(ebase, _CH)], sbuf)
            pltpu.sync_copy(dstf.at[pl.ds(ebase, _CH)], dbuf)
            pltpu.async_copy(h_hbm.at[sbuf], rows, sem).wait()
            for k in range(_CH // 16):
                s16 = sbuf[pl.ds(k * 16, 16)]
                d16 = dbuf[pl.ds(k * 16, 16)]
                sv = plsc.load_gather(asv, [s16 >> 7, s16 & 127])
                dv = plsc.load_gather(adv, [d16 >> 7, d16 & 127])
                al = sv + dv
                al = jnp.where(al >= 0.0, al, 0.2 * al)
                ex = jnp.exp(al - cvec)
                d2 = d16 - off
                inr = (d2 >= 0) & (d2 < _NR)
                exc[pl.ds(k * 16, 16)] = jnp.where(inr, ex, 0.0)
                d2 = jnp.where(d2 < 0, d2 + _NR, d2)
                d2 = jnp.where(d2 >= _NR, d2 - _NR, d2)
                dbuf[pl.ds(k * 16, 16)] = d2

            def edge(e, ecarry):
                z = jnp.zeros((16,), jnp.int32)
                eb = plsc.load_gather(exc, [z + e])
                exb[e, pl.ds(0, 16)] = eb * lane0
                for k in range(_H1 // 16):
                    rows[e, pl.ds(k * 16, 16)] = rows[e, pl.ds(k * 16, 16)] * eb
                return ecarry

            lax.fori_loop(0, _CH, edge, 0)
            pltpu.sync_copy(rows, acc_sh.at[dbuf], add=True)
            pltpu.sync_copy(exb, den_sh.at[dbuf], add=True)
            return carry

        lax.fori_loop(0, _NCH, chunk, 0)
        sc_barrier(1)
        pltpu.sync_copy(acc_sh.at[pl.ds(si * CR, CR)],
                        acc_out.at[pl.ds(off + si * CR, CR)])
        pltpu.sync_copy(den_sh.at[pl.ds(si * CR, CR)],
                        den_out.at[pl.ds(off + si * CR, CR)])

    return f


_SC_EDGE = _make_sc_edge()


def _edge_phase_sc(h, srcr, dstr, as2d, ad2d, cmax):
    asad = jnp.stack([as2d, ad2d])
    return _SC_EDGE(h, srcr, dstr, asad, cmax.reshape(1024))


def kernel(x, edge_index, batch, W1, a_src1, a_dst1, b1,
           W2, a_src2, a_dst2, b2, Wlin, blin):
    x_pad = jnp.zeros((_NPAD, _D_IN), jnp.float32).at[:_N].set(x)
    src = edge_index[0].astype(jnp.int32)
    dst = edge_index[1].astype(jnp.int32)
    batch_pad = jnp.concatenate(
        [batch.astype(jnp.int32), jnp.full((_NPAD - _N,), _G, jnp.int32)])
    h1, as1, ad1, cmax1 = _t1(x_pad, W1, a_src1, a_dst1)
    acc1, den1 = _edge_phase_sc(h1, src, dst, as1, ad1, cmax1)
    h2, as2, ad2, cmax2 = _t2(acc1, den1, b1, W2, a_src2, a_dst2)
    acc2, den2 = _edge_phase_sc(h2, src, dst, as2, ad2, cmax2)
    return _t3(acc2, den2, b2, batch_pad, Wlin, blin)


# speedup vs baseline: 3.4932x; 1.0163x over previous
import functools

import jax
import jax.numpy as jnp
from jax import lax
from jax.experimental import pallas as pl
from jax.experimental.pallas import tpu as pltpu
from jax.experimental.pallas import tpu_sc as plsc

_N = 10000
_E = 320000
_G = 32
_D_IN = 128
_H1 = 128
_H2 = 64
_OUT = 10
_NPAD = 10240
_BT = 1024
_NSUB = 8
_CH = 32
_EPT = _E // _NSUB
_NCH = _EPT // _CH
_NR = _NPAD // 2
_NRP = _NR + 128


def _logit_block(h, asr, adr, as_ref, ad_ref, cmax_ref, i):
    a_s = jnp.sum(h * asr, axis=1)
    a_d = jnp.sum(h * adr, axis=1)
    as_ref[...] = a_s.reshape(_BT // 128, 128)
    ad_ref[...] = a_d.reshape(_BT // 128, 128)

    @pl.when(i == 0)
    def _():
        cmax_ref[...] = jnp.full((8, 128), -1e30, jnp.float32)

    upd = jnp.concatenate(
        [jnp.full((1, 128), jnp.max(a_s)), jnp.full((1, 128), jnp.max(a_d)),
         jnp.full((6, 128), -1e30)], axis=0)
    cmax_ref[...] = jnp.maximum(cmax_ref[...], upd)


def _t1_body(x_ref, w_ref, asr_ref, adr_ref, h_ref, as_ref, ad_ref, cmax_ref):
    i = pl.program_id(0)
    h = jnp.dot(x_ref[...], w_ref[...], preferred_element_type=jnp.float32)
    h_ref[...] = h
    _logit_block(h, asr_ref[...], adr_ref[...], as_ref, ad_ref, cmax_ref, i)


def _t1(x_pad, W1, a_src, a_dst):
    n_blk = _NPAD // _BT
    H = _H1
    return pl.pallas_call(
        _t1_body,
        grid=(n_blk,),
        in_specs=[
            pl.BlockSpec((_BT, _D_IN), lambda i: (i, 0)),
            pl.BlockSpec((_D_IN, H), lambda i: (0, 0)),
            pl.BlockSpec((1, H), lambda i: (0, 0)),
            pl.BlockSpec((1, H), lambda i: (0, 0)),
        ],
        out_specs=[
            pl.BlockSpec((_BT, H), lambda i: (i, 0)),
            pl.BlockSpec((_BT // 128, 128), lambda i: (i, 0)),
            pl.BlockSpec((_BT // 128, 128), lambda i: (i, 0)),
            pl.BlockSpec((8, 128), lambda i: (0, 0)),
        ],
        out_shape=[
            jax.ShapeDtypeStruct((_NPAD, H), jnp.float32),
            jax.ShapeDtypeStruct((_NPAD // 128, 128), jnp.float32),
            jax.ShapeDtypeStruct((_NPAD // 128, 128), jnp.float32),
            jax.ShapeDtypeStruct((8, 128), jnp.float32),
        ],
    )(x_pad, W1, a_src.reshape(1, H), a_dst.reshape(1, H))


def _t2_body(acc_ref, den_ref, b_ref, w_ref, asr_ref, adr_ref,
             h_ref, as_ref, ad_ref, cmax_ref):
    i = pl.program_id(0)
    den = jnp.sum(den_ref[...], axis=1, keepdims=True)
    h1 = jnp.maximum(acc_ref[...] / jnp.maximum(den, 1e-16) + b_ref[...], 0.0)
    h2 = jnp.dot(h1, w_ref[...], preferred_element_type=jnp.float32)
    h_ref[...] = jnp.concatenate(
        [h2, jnp.zeros((_BT, _H1 - _H2), jnp.float32)], axis=1)
    _logit_block(h2, asr_ref[...], adr_ref[...], as_ref, ad_ref, cmax_ref, i)


def _t2(acc, den, b1, W2, a_src2, a_dst2):
    n_blk = _NPAD // _BT
    return pl.pallas_call(
        _t2_body,
        grid=(n_blk,),
        in_specs=[
            pl.BlockSpec((_BT, _H1), lambda i: (i, 0)),
            pl.BlockSpec((_BT, 16), lambda i: (i, 0)),
            pl.BlockSpec((1, _H1), lambda i: (0, 0)),
            pl.BlockSpec((_H1, _H2), lambda i: (0, 0)),
            pl.BlockSpec((1, _H2), lambda i: (0, 0)),
            pl.BlockSpec((1, _H2), lambda i: (0, 0)),
        ],
        out_specs=[
            pl.BlockSpec((_BT, _H1), lambda i: (i, 0)),
            pl.BlockSpec((_BT // 128, 128), lambda i: (i, 0)),
            pl.BlockSpec((_BT // 128, 128), lambda i: (i, 0)),
            pl.BlockSpec((8, 128), lambda i: (0, 0)),
        ],
        out_shape=[
            jax.ShapeDtypeStruct((_NPAD, _H1), jnp.float32),
            jax.ShapeDtypeStruct((_NPAD // 128, 128), jnp.float32),
            jax.ShapeDtypeStruct((_NPAD // 128, 128), jnp.float32),
            jax.ShapeDtypeStruct((8, 128), jnp.float32),
        ],
    )(acc, den, b1.reshape(1, _H1), W2,
      a_src2.reshape(1, _H2), a_dst2.reshape(1, _H2))


def _t3_body(acc_ref, den_ref, b_ref, batch_ref, wl_ref, bl_ref,
             out_ref, p_s, c_s):
    i = pl.program_id(0)
    den = jnp.sum(den_ref[...], axis=1, keepdims=True)
    h2 = jnp.maximum(
        acc_ref[:, :_H2] / jnp.maximum(den, 1e-16) + b_ref[...], 0.0)
    m = batch_ref[...]
    onehot = (m == lax.broadcasted_iota(jnp.int32, (1, _G), 1)
              ).astype(jnp.float32)

    @pl.when(i == 0)
    def _():
        p_s[...] = jnp.zeros((_G, _H2), jnp.float32)
        c_s[...] = jnp.zeros((_G, 128), jnp.float32)

    p_s[...] += lax.dot_general(onehot, h2, (((0,), (0,)), ((), ())),
                                preferred_element_type=jnp.float32)
    c_s[...] += lax.dot_general(onehot, jnp.ones((_BT, 128), jnp.float32),
                                (((0,), (0,)), ((), ())),
                                preferred_element_type=jnp.float32)

    @pl.when(i == pl.num_programs(0) - 1)
    def _():
        cnt = c_s[:, 0:1]
        pooled = p_s[...] / jnp.maximum(cnt, 1.0)
        out_ref[...] = jnp.dot(pooled, wl_ref[...],
                               preferred_element_type=jnp.float32) + bl_ref[...]


def _t3(acc2, den2, b2, batch_pad, Wlin, blin):
    n_blk = _NPAD // _BT
    return pl.pallas_call(
        _t3_body,
        grid=(n_blk,),
        in_specs=[
            pl.BlockSpec((_BT, _H1), lambda i: (i, 0)),
            pl.BlockSpec((_BT, 16), lambda i: (i, 0)),
            pl.BlockSpec((1, _H2), lambda i: (0, 0)),
            pl.BlockSpec((_BT, 1), lambda i: (i, 0)),
            pl.BlockSpec((_H2, _OUT), lambda i: (0, 0)),
            pl.BlockSpec((1, _OUT), lambda i: (0, 0)),
        ],
        out_specs=pl.BlockSpec((_G, _OUT), lambda i: (0, 0)),
        out_shape=jax.ShapeDtypeStruct((_G, _OUT), jnp.float32),
        scratch_shapes=[
            pltpu.VMEM((_G, _H2), jnp.float32),
            pltpu.VMEM((_G, 128), jnp.float32),
        ],
    )(acc2, den2, b2.reshape(1, _H2), batch_pad.reshape(_NPAD, 1),
      Wlin, blin.reshape(1, _OUT))


def _make_sc_edge():
    mesh = plsc.VectorSubcoreMesh(core_axis_name="c", subcore_axis_name="s",
                                  num_subcores=_NSUB)
    CR = _NR // _NSUB

    @functools.partial(
        pl.kernel,
        out_type=[
            jax.ShapeDtypeStruct((_NPAD, _H1), jnp.float32),
            jax.ShapeDtypeStruct((_NPAD, 16), jnp.float32),
        ],
        mesh=mesh,
        compiler_params=pltpu.CompilerParams(needs_layout_passes=False),
        scratch_types=[
            pltpu.VMEM_SHARED((_NR, _H1), jnp.float32),
            pltpu.VMEM_SHARED((_NR, 16), jnp.float32),
            pltpu.VMEM((_CH,), jnp.int32),
            pltpu.VMEM((_CH,), jnp.int32),
            pltpu.VMEM((_NPAD // 128, 128), jnp.float32),
            pltpu.VMEM((_NPAD // 128, 128), jnp.float32),
            pltpu.VMEM((16,), jnp.float32),
            pltpu.VMEM((16,), jnp.float32),
            pltpu.VMEM((_CH, _H1), jnp.float32),
            pltpu.VMEM((_CH,), jnp.float32),
            pltpu.VMEM((_CH, 16), jnp.float32),
            pltpu.SMEM((2,), jnp.int32),
            pltpu.SemaphoreType.DMA,
        ],
    )
    def f(h_hbm, srcf, dstf, asad, cflat, acc_out, den_out,
          acc_sh, den_sh, sbuf, dbuf, asv, adv, ca, cd, rows, exc, exb,
          cnt, sem):
        ci = lax.axis_index("c")
        si = lax.axis_index("s")

        @pl.when(si == 0)
        def _():
            cnt[0] = 0
            cnt[1] = 0

        def sc_barrier(phase):
            plsc.fetch_and_add(cnt.at[phase], 1, subcore_id=0)
            lax.while_loop(
                lambda v: v < _NSUB,
                lambda v: plsc.fetch_and_add(cnt.at[phase], 0, subcore_id=0),
                0)

        pltpu.sync_copy(asad.at[0], asv)
        pltpu.sync_copy(asad.at[1], adv)
        pltpu.sync_copy(cflat.at[pl.ds(0, 16)], ca)
        pltpu.sync_copy(cflat.at[pl.ds(128, 16)], cd)
        s = ca[...] + cd[...]
        cvec = jnp.where(s >= 0.0, s, 0.2 * s)
        lane0 = jnp.where(lax.iota(jnp.int32, 16) == 0, 1.0, 0.0)
        z16 = jnp.zeros((16,), jnp.float32)
        off = ci * _NR

        def zrow_body(r, carry):
            for k in range(_H1 // 16):
                rows[r, pl.ds(k * 16, 16)] = z16
            exb[r, pl.ds(0, 16)] = z16
            return carry

        lax.fori_loop(0, _CH, zrow_body, 0)
        base = si * CR
        for j in range(CR // _CH):
            pltpu.sync_copy(rows, acc_sh.at[pl.ds(base + j * _CH, _CH)])
            pltpu.sync_copy(exb, den_sh.at[pl.ds(base + j * _CH, _CH)])
        sc_barrier(0)

        def chunk(cidx, carry):
            ebase = (si * _NCH + cidx) * _CH
            pltpu.sync_copy(srcf.at[pl.ds(ebase, _CH)], sbuf)
            pltpu.sync_copy(dstf.at[pl.ds(ebase, _CH)], dbuf)
            gat = pltpu.async_copy(h_hbm.at[sbuf], rows, sem)
            for k in range(_CH // 16):
                s16 = sbuf[pl.ds(k * 16, 16)]
                d16 = dbuf[pl.ds(k * 16, 16)]
                sv = plsc.load_gather(asv, [s16 >> 7, s16 & 127])
                dv = plsc.load_gather(adv, [d16 >> 7, d16 & 127])
                al = sv + dv
                al = jnp.where(al >= 0.0, al, 0.2 * al)
                ex = jnp.exp(al - cvec)
                d2 = d16 - off
                inr = (d2 >= 0) & (d2 < _NR)
                exc[pl.ds(k * 16, 16)] = jnp.where(inr, ex, 0.0)
                d2 = jnp.where(d2 < 0, d2 + _NR, d2)
                d2 = jnp.where(d2 >= _NR, d2 - _NR, d2)
                dbuf[pl.ds(k * 16, 16)] = d2
            gat.wait()

            def edge(e, ecarry):
                z = jnp.zeros((16,), jnp.int32)
                eb = plsc.load_gather(exc, [z + e])
                exb[e, pl.ds(0, 16)] = eb * lane0
                for k in range(_H1 // 16):
                    rows[e, pl.ds(k * 16, 16)] = rows[e, pl.ds(k * 16, 16)] * eb
                return ecarry

            lax.fori_loop(0, _CH, edge, 0)
            pltpu.sync_copy(rows, acc_sh.at[dbuf], add=True)
            pltpu.sync_copy(exb, den_sh.at[dbuf], add=True)
            return carry

        lax.fori_loop(0, _NCH, chunk, 0)
        sc_barrier(1)
        pltpu.sync_copy(acc_sh.at[pl.ds(si * CR, CR)],
                        acc_out.at[pl.ds(off + si * CR, CR)])
        pltpu.sync_copy(den_sh.at[pl.ds(si * CR, CR)],
                        den_out.at[pl.ds(off + si * CR, CR)])

    return f


_SC_EDGE = _make_sc_edge()


def _edge_phase_sc(h, srcr, dstr, as2d, ad2d, cmax):
    asad = jnp.stack([as2d, ad2d])
    return _SC_EDGE(h, srcr, dstr, asad, cmax.reshape(1024))


def kernel(x, edge_index, batch, W1, a_src1, a_dst1, b1,
           W2, a_src2, a_dst2, b2, Wlin, blin):
    x_pad = jnp.zeros((_NPAD, _D_IN), jnp.float32).at[:_N].set(x)
    src = edge_index[0].astype(jnp.int32)
    dst = edge_index[1].astype(jnp.int32)
    batch_pad = jnp.concatenate(
        [batch.astype(jnp.int32), jnp.full((_NPAD - _N,), _G, jnp.int32)])
    h1, as1, ad1, cmax1 = _t1(x_pad, W1, a_src1, a_dst1)
    acc1, den1 = _edge_phase_sc(h1, src, dst, as1, ad1, cmax1)
    h2, as2, ad2, cmax2 = _t2(acc1, den1, b1, W2, a_src2, a_dst2)
    acc2, den2 = _edge_phase_sc(h2, src, dst, as2, ad2, cmax2)
    return _t3(acc2, den2, b2, batch_pad, Wlin, blin)
